# baseline (speedup 1.0000x reference)
import functools

import jax
import jax.numpy as jnp
from jax.experimental import pallas as pl

_RC2 = 36.0
_EPS = 1e-12


def _body(d_rep_ref, p_rep_ref, d_pl_ref, pos_ref, vrow_ref, vcol_ref,
          vrep_ref, vec_ref, sod_ref, mask_ref, *, N):
    N3 = 3 * N
    vcolb = vcol_ref[0] > 0.0

    d_row = d_rep_ref[0, 0]
    vec = d_row - p_rep_ref[0]
    sq = vec * vec
    t = sq + pltpu_roll(sq, -1) + pltpu_roll(sq, -2)
    lane = jax.lax.broadcasted_iota(jnp.int32, (N, N3), 1)
    t2 = jnp.where(lane % 3 == 0, t, 0.0)
    s_rep = t2 + pltpu_roll(t2, 1) + pltpu_roll(t2, 2)
    vrepb = vrep_ref[0] > 0.0
    m_rep = (s_rep < _RC2) & (s_rep > _EPS) & vrepb & vcolb
    vec_ref[0, 0] = jnp.where(m_rep, vec, 0.0)

    dp = d_pl_ref[0, 0]
    p = pos_ref[0]
    vx = dp[0:1, :] - p[:, 0:1]
    vy = dp[1:2, :] - p[:, 1:2]
    vz = dp[2:3, :] - p[:, 2:3]
    sod = vx * vx + vy * vy + vz * vz
    vrowb = vrow_ref[0] > 0.0
    m = (sod < _RC2) & (sod > _EPS) & vrowb & vcolb
    sod_ref[0, 0] = jnp.where(m, sod, 0.0)
    mask_ref[0, 0] = m


def pltpu_roll(x, shift):
    return jnp.roll(x, shift, axis=1)


@jax.jit
def kernel(pos, cel, sft_cel, ent):
    B, N, _ = pos.shape
    S = sft_cel.shape[0]
    N3 = 3 * N
    f32 = pos.dtype

    sft_xyz = jnp.einsum('sk,bkl->bsl', sft_cel.astype(f32), cel)
    d = pos[:, None, :, :] + sft_xyz[:, :, None, :]
    d_rep = d.reshape(B, S, 1, N3)
    d_pl = d.transpose(0, 1, 3, 2)
    p_rep = jnp.tile(pos, (1, 1, N))
    validf = (ent > 0).astype(f32)
    vrow = validf.reshape(B, 1, N)
    vcol = validf.reshape(B, N, 1)
    vrep = jnp.repeat(validf, 3, axis=1).reshape(B, 1, N3)

    grid = (B, S)
    vec_out, sod_out, mask_out = pl.pallas_call(
        functools.partial(_body, N=N),
        grid=grid,
        in_specs=[
            pl.BlockSpec((1, 1, 1, N3), lambda b, s: (b, s, 0, 0)),
            pl.BlockSpec((1, N, N3), lambda b, s: (b, 0, 0)),
            pl.BlockSpec((1, 1, 3, N), lambda b, s: (b, s, 0, 0)),
            pl.BlockSpec((1, N, 3), lambda b, s: (b, 0, 0)),
            pl.BlockSpec((1, 1, N), lambda b, s: (b, 0, 0)),
            pl.BlockSpec((1, N, 1), lambda b, s: (b, 0, 0)),
            pl.BlockSpec((1, 1, N3), lambda b, s: (b, 0, 0)),
        ],
        out_specs=[
            pl.BlockSpec((1, 1, N, N3), lambda b, s: (b, s, 0, 0)),
            pl.BlockSpec((1, 1, N, N), lambda b, s: (b, s, 0, 0)),
            pl.BlockSpec((1, 1, N, N), lambda b, s: (b, s, 0, 0)),
        ],
        out_shape=[
            jax.ShapeDtypeStruct((B, S, N, N3), f32),
            jax.ShapeDtypeStruct((B, S, N, N), f32),
            jax.ShapeDtypeStruct((B, S, N, N), jnp.bool_),
        ],
    )(d_rep, p_rep, d_pl, pos, vrow, vcol, vrep)

    return vec_out.reshape(B, S, N, N, 3), sod_out, mask_out


# speedup vs baseline: 5.8740x; 5.8740x over previous
import functools

import jax
import jax.numpy as jnp
from jax.experimental import pallas as pl

_RC2 = 36.0
_EPS = 1e-12


def _body(d_pl_ref, pos_ref, vrow_ref, vcol_ref,
          vec_ref, sod_ref, mask_ref):
    dp = d_pl_ref[0, 0]
    p = pos_ref[0]
    vx = dp[0:1, :] - p[:, 0:1]
    vy = dp[1:2, :] - p[:, 1:2]
    vz = dp[2:3, :] - p[:, 2:3]
    sod = vx * vx + vy * vy + vz * vz
    vrowb = vrow_ref[0] > 0.0
    vcolb = vcol_ref[0] > 0.0
    m = (sod < _RC2) & (sod > _EPS) & vrowb & vcolb
    sod_ref[0, 0] = jnp.where(m, sod, 0.0)
    mask_ref[0, 0] = m
    vec_ref[0, 0, 0] = jnp.where(m, vx, 0.0)
    vec_ref[0, 0, 1] = jnp.where(m, vy, 0.0)
    vec_ref[0, 0, 2] = jnp.where(m, vz, 0.0)


@jax.jit
def kernel(pos, cel, sft_cel, ent):
    B, N, _ = pos.shape
    S = sft_cel.shape[0]
    f32 = pos.dtype

    sft_xyz = jnp.einsum('sk,bkl->bsl', sft_cel.astype(f32), cel)
    d = pos[:, None, :, :] + sft_xyz[:, :, None, :]
    d_pl = d.transpose(0, 1, 3, 2)
    validf = (ent > 0).astype(f32)
    vrow = validf.reshape(B, 1, N)
    vcol = validf.reshape(B, N, 1)

    grid = (B, S)
    vec_out, sod_out, mask_out = pl.pallas_call(
        _body,
        grid=grid,
        in_specs=[
            pl.BlockSpec((1, 1, 3, N), lambda b, s: (b, s, 0, 0)),
            pl.BlockSpec((1, N, 3), lambda b, s: (b, 0, 0)),
            pl.BlockSpec((1, 1, N), lambda b, s: (b, 0, 0)),
            pl.BlockSpec((1, N, 1), lambda b, s: (b, 0, 0)),
        ],
        out_specs=[
            pl.BlockSpec((1, 1, 3, N, N), lambda b, s: (b, s, 0, 0, 0)),
            pl.BlockSpec((1, 1, N, N), lambda b, s: (b, s, 0, 0)),
            pl.BlockSpec((1, 1, N, N), lambda b, s: (b, s, 0, 0)),
        ],
        out_shape=[
            jax.ShapeDtypeStruct((B, S, 3, N, N), f32),
            jax.ShapeDtypeStruct((B, S, N, N), f32),
            jax.ShapeDtypeStruct((B, S, N, N), jnp.bool_),
        ],
    )(d_pl, pos, vrow, vcol)

    return vec_out.transpose(0, 1, 3, 4, 2), sod_out, mask_out


# speedup vs baseline: 6.2090x; 1.0570x over previous
import functools

import jax
import jax.numpy as jnp
from jax.experimental import pallas as pl

_RC2 = 36.0
_EPS = 1e-12


def _body(d_pl_ref, pos_ref, vrow_ref, vcol_ref,
          vec_ref, sod_ref, mask_ref, *, SC):
    p = pos_ref[0]
    vrowb = vrow_ref[0] > 0.0
    vcolb = vcol_ref[0] > 0.0
    for k in range(SC):
        dp = d_pl_ref[0, k]
        vx = dp[0:1, :] - p[:, 0:1]
        vy = dp[1:2, :] - p[:, 1:2]
        vz = dp[2:3, :] - p[:, 2:3]
        sod = vx * vx + vy * vy + vz * vz
        m = (sod < _RC2) & (sod > _EPS) & vrowb & vcolb
        sod_ref[0, k] = jnp.where(m, sod, 0.0)
        mask_ref[0, k] = m
        vec_ref[0, k, 0] = jnp.where(m, vx, 0.0)
        vec_ref[0, k, 1] = jnp.where(m, vy, 0.0)
        vec_ref[0, k, 2] = jnp.where(m, vz, 0.0)


@jax.jit
def kernel(pos, cel, sft_cel, ent):
    B, N, _ = pos.shape
    S = sft_cel.shape[0]
    f32 = pos.dtype

    sft_xyz = jnp.einsum('sk,bkl->bsl', sft_cel.astype(f32), cel)
    d = pos[:, None, :, :] + sft_xyz[:, :, None, :]
    d_pl = d.transpose(0, 1, 3, 2)
    validf = (ent > 0).astype(f32)
    vrow = validf.reshape(B, 1, N)
    vcol = validf.reshape(B, N, 1)

    SC = 3
    grid = (B, S // SC)
    vec_out, sod_out, mask_out = pl.pallas_call(
        functools.partial(_body, SC=SC),
        grid=grid,
        in_specs=[
            pl.BlockSpec((1, SC, 3, N), lambda b, s: (b, s, 0, 0)),
            pl.BlockSpec((1, N, 3), lambda b, s: (b, 0, 0)),
            pl.BlockSpec((1, 1, N), lambda b, s: (b, 0, 0)),
            pl.BlockSpec((1, N, 1), lambda b, s: (b, 0, 0)),
        ],
        out_specs=[
            pl.BlockSpec((1, SC, 3, N, N), lambda b, s: (b, s, 0, 0, 0)),
            pl.BlockSpec((1, SC, N, N), lambda b, s: (b, s, 0, 0)),
            pl.BlockSpec((1, SC, N, N), lambda b, s: (b, s, 0, 0)),
        ],
        out_shape=[
            jax.ShapeDtypeStruct((B, S, 3, N, N), f32),
            jax.ShapeDtypeStruct((B, S, N, N), f32),
            jax.ShapeDtypeStruct((B, S, N, N), jnp.bool_),
        ],
    )(d_pl, pos, vrow, vcol)

    return vec_out.transpose(0, 1, 3, 4, 2), sod_out, mask_out
